# baseline (speedup 1.0000x reference)
import functools

import jax
import jax.numpy as jnp
from jax import lax
from jax.experimental import pallas as pl
from jax.experimental.pallas import tpu as pltpu
from jax.experimental.pallas import tpu_sc as plsc

_NBUF = 2
_GPP = 8
_RPS = 8
_SLAB = 8


@functools.lru_cache(maxsize=None)
def _make_sc_kernel(B, D):
    info = plsc.get_sparse_core_info()
    NC, NS, L = info.num_cores, info.num_subcores, info.num_lanes
    NW = NC * NS
    b_per_w = B // NW
    n_groups = b_per_w // L
    n_phases = n_groups // _GPP
    rpp = _GPP * L

    mesh = plsc.VectorSubcoreMesh(core_axis_name="c", subcore_axis_name="s")

    @functools.partial(
        pl.kernel,
        mesh=mesh,
        out_type=jax.ShapeDtypeStruct((B,), jnp.float32),
        compiler_params=pltpu.CompilerParams(needs_layout_passes=False),
        scratch_types=[
            pltpu.VMEM((b_per_w,), jnp.int32),
            pltpu.VMEM((b_per_w,), jnp.int32),
            pltpu.VMEM((_NBUF * rpp, D), jnp.float32),
            pltpu.VMEM((_NBUF * rpp, D), jnp.float32),
            pltpu.VMEM((b_per_w,), jnp.float32),
            pltpu.SemaphoreType.DMA,
        ],
    )
    def sc_kernel(uidx_hbm, iidx_hbm, utab_hbm, itab_hbm, out_hbm,
                  uidx_v, iidx_v, urows, irows, out_v, sem):
        wid = lax.axis_index("s") * NC + lax.axis_index("c")
        base = wid * b_per_w

        pltpu.sync_copy(uidx_hbm.at[pl.ds(base, b_per_w)], uidx_v)
        pltpu.sync_copy(iidx_hbm.at[pl.ds(base, b_per_w)], iidx_v)

        def enqueue_phase(p):
            slot = lax.rem(p, _NBUF) * rpp

            def enqueue_grp(gl, carry):
                iv_u = uidx_v[pl.ds(p * rpp + gl * L, L)]
                iv_i = iidx_v[pl.ds(p * rpp + gl * L, L)]
                for l in range(L):
                    pltpu.async_copy(
                        utab_hbm.at[iv_u[l]], urows.at[slot + gl * L + l], sem)
                    pltpu.async_copy(
                        itab_hbm.at[iv_i[l]], irows.at[slot + gl * L + l], sem)
                return carry

            lax.fori_loop(0, _GPP, enqueue_grp, 0)

        def drain_phase():
            pltpu.make_async_copy(
                utab_hbm.at[pl.ds(0, rpp)], urows.at[pl.ds(0, rpp)], sem
            ).wait()
            pltpu.make_async_copy(
                itab_hbm.at[pl.ds(0, rpp)], irows.at[pl.ds(0, rpp)], sem
            ).wait()

        enqueue_phase(0)

        def phase_body(p, carry):
            @pl.when(p + 1 < n_phases)
            def _():
                enqueue_phase(p + 1)

            drain_phase()

            slot = lax.rem(p, _NBUF) * rpp

            def compute_grp(gl, carry2):
                rows = slot + gl * L + lax.iota(jnp.int32, L)
                acc = jnp.zeros((L,), jnp.float32)
                for d in range(D):
                    cols = jnp.full((L,), d, jnp.int32)
                    u = plsc.load_gather(urows, [rows, cols])
                    v = plsc.load_gather(irows, [rows, cols])
                    acc = acc + u * v
                out_v[pl.ds(p * rpp + gl * L, L)] = acc
                return carry2

            lax.fori_loop(0, _GPP, compute_grp, 0)
            return carry

        lax.fori_loop(0, n_phases, phase_body, 0)

        pltpu.sync_copy(out_v, out_hbm.at[pl.ds(base, b_per_w)])

    return sc_kernel


@functools.lru_cache(maxsize=None)
def _make_tc_kernel(B, D):
    n_steps = B // _RPS

    def body(idx_ref, *refs):
        i = pl.program_id(0)
        urefs = refs[:_RPS]
        irefs = refs[_RPS:2 * _RPS]
        out_ref = refs[2 * _RPS]
        rows = []
        for k in range(_RPS):
            ru = idx_ref[_RPS * i + k] % _SLAB
            ri = idx_ref[B + _RPS * i + k] % _SLAB
            u = urefs[k][pl.ds(ru, 1), :]
            v = irefs[k][pl.ds(ri, 1), :]
            rows.append(u * v)
        prods = jnp.concatenate(rows, axis=0)
        out_ref[...] = jnp.sum(prods, axis=1, keepdims=True)

    def u_spec(k):
        return pl.BlockSpec(
            (_SLAB, D), lambda i, idx: (idx[_RPS * i + k] // _SLAB, 0))

    def i_spec(k):
        return pl.BlockSpec(
            (_SLAB, D), lambda i, idx: (idx[B + _RPS * i + k] // _SLAB, 0))

    grid_spec = pltpu.PrefetchScalarGridSpec(
        num_scalar_prefetch=1,
        grid=(n_steps,),
        in_specs=[u_spec(k) for k in range(_RPS)]
        + [i_spec(k) for k in range(_RPS)],
        out_specs=pl.BlockSpec((_RPS, 1), lambda i, idx: (i, 0)),
    )
    return pl.pallas_call(
        body,
        grid_spec=grid_spec,
        out_shape=jax.ShapeDtypeStruct((B, 1), jnp.float32),
    )


_TC_ROWS = 4096


def kernel(user_idx, item_idx, user_table, item_table):
    B = user_idx.shape[0]
    D = user_table.shape[1]
    uidx = user_idx.astype(jnp.int32)
    iidx = item_idx.astype(jnp.int32)

    bt = _TC_ROWS
    sc_out = _make_sc_kernel(B - bt, D)(
        uidx[bt:], iidx[bt:], user_table, item_table)

    tc_idx = jnp.concatenate([uidx[:bt], iidx[:bt]])
    tabs = [user_table] * _RPS + [item_table] * _RPS
    tc_out = _make_tc_kernel(bt, D)(tc_idx, *tabs)

    return jnp.concatenate([tc_out, sc_out.reshape(B - bt, 1)], axis=0)


# speedup vs baseline: 1.4182x; 1.4182x over previous
import functools

import jax
import jax.numpy as jnp
from jax import lax
from jax.experimental import pallas as pl
from jax.experimental.pallas import tpu as pltpu
from jax.experimental.pallas import tpu_sc as plsc

_NBUF = 2
_GPP = 8


@functools.lru_cache(maxsize=None)
def _make_sc_kernel(B, D):
    info = plsc.get_sparse_core_info()
    NC, NS, L = info.num_cores, info.num_subcores, info.num_lanes
    NW = NC * NS
    b_per_w = B // NW
    n_groups = b_per_w // L
    n_phases = n_groups // _GPP
    rpp = _GPP * L

    mesh = plsc.VectorSubcoreMesh(core_axis_name="c", subcore_axis_name="s")

    @functools.partial(
        pl.kernel,
        mesh=mesh,
        out_type=jax.ShapeDtypeStruct((B,), jnp.float32),
        compiler_params=pltpu.CompilerParams(needs_layout_passes=False),
        scratch_types=[
            pltpu.VMEM((b_per_w,), jnp.int32),
            pltpu.VMEM((b_per_w,), jnp.int32),
            pltpu.VMEM((_NBUF * rpp, D), jnp.float32),
            pltpu.VMEM((_NBUF * rpp, D), jnp.float32),
            pltpu.VMEM((b_per_w,), jnp.float32),
            pltpu.SemaphoreType.DMA,
        ],
    )
    def sc_kernel(uidx_hbm, iidx_hbm, utab_hbm, itab_hbm, out_hbm,
                  uidx_v, iidx_v, urows, irows, out_v, sem):
        wid = lax.axis_index("s") * NC + lax.axis_index("c")
        base = wid * b_per_w

        pltpu.sync_copy(uidx_hbm.at[pl.ds(base, b_per_w)], uidx_v)
        pltpu.sync_copy(iidx_hbm.at[pl.ds(base, b_per_w)], iidx_v)

        def enqueue_phase(p):
            slot = lax.rem(p, _NBUF) * rpp

            def enqueue_grp(gl, carry):
                iv_u = uidx_v[pl.ds(p * rpp + gl * L, L)]
                iv_i = iidx_v[pl.ds(p * rpp + gl * L, L)]
                for l in range(L):
                    pltpu.async_copy(
                        utab_hbm.at[iv_u[l]], urows.at[slot + gl * L + l], sem)
                    pltpu.async_copy(
                        itab_hbm.at[iv_i[l]], irows.at[slot + gl * L + l], sem)
                return carry

            lax.fori_loop(0, _GPP, enqueue_grp, 0)

        def drain_phase():
            pltpu.make_async_copy(
                utab_hbm.at[pl.ds(0, rpp)], urows.at[pl.ds(0, rpp)], sem
            ).wait()
            pltpu.make_async_copy(
                itab_hbm.at[pl.ds(0, rpp)], irows.at[pl.ds(0, rpp)], sem
            ).wait()

        enqueue_phase(0)

        def phase_body(p, carry):
            @pl.when(p + 1 < n_phases)
            def _():
                enqueue_phase(p + 1)

            drain_phase()

            slot = lax.rem(p, _NBUF) * rpp

            def compute_grp(gl, carry2):
                rows = slot + gl * L + lax.iota(jnp.int32, L)
                acc = jnp.zeros((L,), jnp.float32)
                for d in range(D):
                    cols = jnp.full((L,), d, jnp.int32)
                    u = plsc.load_gather(urows, [rows, cols])
                    v = plsc.load_gather(irows, [rows, cols])
                    acc = acc + u * v
                out_v[pl.ds(p * rpp + gl * L, L)] = acc
                return carry2

            lax.fori_loop(0, _GPP, compute_grp, 0)
            return carry

        lax.fori_loop(0, n_phases, phase_body, 0)

        pltpu.sync_copy(out_v, out_hbm.at[pl.ds(base, b_per_w)])

    return sc_kernel


def kernel(user_idx, item_idx, user_table, item_table):
    B = user_idx.shape[0]
    D = user_table.shape[1]
    uidx = user_idx.astype(jnp.int32)
    iidx = item_idx.astype(jnp.int32)
    out = _make_sc_kernel(B, D)(uidx, iidx, user_table, item_table)
    return out.reshape(B, 1)
